# initial kernel scaffold (unmeasured)
import jax
import jax.numpy as jnp
from jax import lax
from jax.experimental import pallas as pl
from jax.experimental.pallas import tpu as pltpu

N_DEV = 4
M_PER = 2048
K_PER = 2048
N_OUT = 4096
BN = 1024


def _a2a(x):

    def body(x_ref, xg_ref, send_sems, recv_sems, local_sem):
        my = lax.axis_index("i")

        barrier_sem = pltpu.get_barrier_semaphore()
        for d in range(1, N_DEV):
            peer = lax.rem(my + d, N_DEV)
            pl.semaphore_signal(
                barrier_sem, inc=1,
                device_id=(peer,), device_id_type=pl.DeviceIdType.MESH,
            )
        pl.semaphore_wait(barrier_sem, N_DEV - 1)

        local = pltpu.make_async_copy(
            x_ref.at[pl.ds(my * M_PER, M_PER), :], xg_ref.at[my], local_sem
        )
        local.start()

        rdmas = []
        for d in range(1, N_DEV):
            peer = lax.rem(my + d, N_DEV)
            rdma = pltpu.make_async_remote_copy(
                src_ref=x_ref.at[pl.ds(peer * M_PER, M_PER), :],
                dst_ref=xg_ref.at[my],
                send_sem=send_sems.at[d - 1],
                recv_sem=recv_sems.at[d - 1],
                device_id=(peer,),
                device_id_type=pl.DeviceIdType.MESH,
            )
            rdma.start()
            rdmas.append(rdma)

        local.wait()
        for rdma in rdmas:
            rdma.wait()

    return pl.pallas_call(
        body,
        out_shape=jax.ShapeDtypeStruct((N_DEV, M_PER, K_PER), jnp.float32),
        in_specs=[pl.BlockSpec(memory_space=pltpu.ANY)],
        out_specs=pl.BlockSpec(memory_space=pltpu.ANY),
        scratch_shapes=[
            pltpu.SemaphoreType.DMA((N_DEV - 1,)),
            pltpu.SemaphoreType.DMA((N_DEV - 1,)),
            pltpu.SemaphoreType.DMA,
        ],
        compiler_params=pltpu.CompilerParams(collective_id=0),
    )(x)


def _gemm_silu(xg, w):
    n_tiles = N_OUT // BN

    def body(xg_ref, w_ref, o_ref, acc_ref):
        k = pl.program_id(1)

        @pl.when(k == 0)
        def _():
            acc_ref[...] = jnp.zeros_like(acc_ref)

        acc_ref[...] += jnp.dot(
            xg_ref[0], w_ref[...], preferred_element_type=jnp.float32
        )

        @pl.when(k == N_DEV - 1)
        def _():
            y = acc_ref[...]
            o_ref[...] = y * jax.nn.sigmoid(y)

    return pl.pallas_call(
        body,
        grid=(n_tiles, N_DEV),
        in_specs=[
            pl.BlockSpec((1, M_PER, K_PER), lambda n, k: (k, 0, 0)),
            pl.BlockSpec((K_PER, BN), lambda n, k: (k, n)),
        ],
        out_specs=pl.BlockSpec((M_PER, BN), lambda n, k: (0, n)),
        out_shape=jax.ShapeDtypeStruct((M_PER, N_OUT), jnp.float32),
        scratch_shapes=[pltpu.VMEM((M_PER, BN), jnp.float32)],
        compiler_params=pltpu.CompilerParams(
            dimension_semantics=("parallel", "arbitrary"),
        ),
    )(xg, w)


def kernel(x, w_mat):
    xg = _a2a(x)
    return _gemm_silu(xg, w_mat)


# baseline (device time: 694047 ns/iter reference)
import jax
import jax.numpy as jnp
from jax import lax
from jax.experimental import pallas as pl
from jax.experimental.pallas import tpu as pltpu

N_DEV = 4
M_PER = 2048
K_PER = 2048
N_OUT = 4096
BN = 1024


def _a2a(x):

    def body(x_ref, xg_ref, send_sems, recv_sems, local_sem):
        my = lax.axis_index("i")

        barrier_sem = pltpu.get_barrier_semaphore()
        for d in range(1, N_DEV):
            peer = lax.rem(my + d, N_DEV)
            pl.semaphore_signal(
                barrier_sem, inc=1,
                device_id=(peer,), device_id_type=pl.DeviceIdType.MESH,
            )
        pl.semaphore_wait(barrier_sem, N_DEV - 1)

        local = pltpu.make_async_copy(
            x_ref.at[pl.ds(my * M_PER, M_PER), :], xg_ref.at[my], local_sem
        )
        local.start()

        rdmas = []
        for d in range(1, N_DEV):
            peer = lax.rem(my + d, N_DEV)
            rdma = pltpu.make_async_remote_copy(
                src_ref=x_ref.at[pl.ds(peer * M_PER, M_PER), :],
                dst_ref=xg_ref.at[my],
                send_sem=send_sems.at[d - 1],
                recv_sem=recv_sems.at[d - 1],
                device_id=(peer,),
                device_id_type=pl.DeviceIdType.MESH,
            )
            rdma.start()
            rdmas.append(rdma)

        local.wait()
        for rdma in rdmas:
            rdma.wait()

    return pl.pallas_call(
        body,
        out_shape=jax.ShapeDtypeStruct((N_DEV, M_PER, K_PER), jnp.float32),
        in_specs=[pl.BlockSpec(memory_space=pl.ANY)],
        out_specs=pl.BlockSpec(memory_space=pl.ANY),
        scratch_shapes=[
            pltpu.SemaphoreType.DMA((N_DEV - 1,)),
            pltpu.SemaphoreType.DMA((N_DEV - 1,)),
            pltpu.SemaphoreType.DMA,
        ],
        compiler_params=pltpu.CompilerParams(collective_id=0),
    )(x)


BK = 1024


def _gemm_silu(xg, w):
    n_tiles = N_OUT // BN
    k_tiles = N_DEV * (K_PER // BK)

    def body(xg_ref, w_ref, o_ref, acc_ref):
        k = pl.program_id(1)

        @pl.when(k == 0)
        def _():
            acc_ref[...] = jnp.zeros_like(acc_ref)

        acc_ref[...] += jnp.dot(
            xg_ref[0], w_ref[...], preferred_element_type=jnp.float32
        )

        @pl.when(k == k_tiles - 1)
        def _():
            y = acc_ref[...]
            o_ref[...] = y * jax.nn.sigmoid(y)

    kb = K_PER // BK

    return pl.pallas_call(
        body,
        grid=(n_tiles, k_tiles),
        in_specs=[
            pl.BlockSpec((1, M_PER, BK), lambda n, k: (k // kb, 0, k % kb)),
            pl.BlockSpec((BK, BN), lambda n, k: (k, n)),
        ],
        out_specs=pl.BlockSpec((M_PER, BN), lambda n, k: (0, n)),
        out_shape=jax.ShapeDtypeStruct((M_PER, N_OUT), jnp.float32),
        scratch_shapes=[pltpu.VMEM((M_PER, BN), jnp.float32)],
        compiler_params=pltpu.CompilerParams(
            dimension_semantics=("parallel", "arbitrary"),
            vmem_limit_bytes=60 * 1024 * 1024,
        ),
    )(xg, w)


def kernel(x, w_mat):
    xg = _a2a(x)
    return _gemm_silu(xg, w_mat)
